# Initial kernel scaffold; baseline (speedup 1.0000x reference)
import jax
import jax.numpy as jnp
from jax.experimental import pallas as pl


def kernel(x, iInd, jInd):
    raise NotImplementedError("write your pallas kernel here")


# speedup vs baseline: 131.5631x; 131.5631x over previous
import functools

import jax
import jax.numpy as jnp
from jax import lax
from jax.experimental import pallas as pl
from jax.experimental.pallas import tpu as pltpu
from jax.experimental.pallas import tpu_sc as plsc

N = 100000
NP = 100096
E = 1600000
DH = 16
NC = 2
NS = 16
B = 80
EPT = E // NS
NCH = EPT // B
RPT = NP // NS
ZB = 16
NZ = RPT // ZB

_mesh = plsc.VectorSubcoreMesh(
    core_axis_name="c", subcore_axis_name="s", num_cores=NC, num_subcores=NS
)


@functools.partial(
    pl.kernel,
    out_type=jax.ShapeDtypeStruct((NC, NP, DH), jnp.float32),
    mesh=_mesh,
    scratch_types=[
        pltpu.VMEM((B,), jnp.int32),
        pltpu.VMEM((B,), jnp.int32),
        pltpu.VMEM((B, DH), jnp.float32),
        pltpu.VMEM((B, DH), jnp.float32),
        pltpu.VMEM((B, DH), jnp.float32),
        pltpu.VMEM((B, DH), jnp.float32),
        pltpu.VMEM((ZB, DH), jnp.float32),
        pltpu.VMEM_SHARED((NP, DH), jnp.float32),
        pltpu.SemaphoreType.DMA,
        pltpu.SemaphoreType.DMA,
    ],
    compiler_params=pltpu.CompilerParams(use_tc_tiling_on_sc=False),
)
def _vector_graph_sc(xh, ii, jj, out, ibuf, jbuf, bi, bj, bd, bnd, zbuf,
                     acc, sem1, sem2):
    c = lax.axis_index("c")
    s = lax.axis_index("s")

    zero_v = jnp.zeros((DH,), jnp.float32)
    for r in range(ZB):
        zbuf[r, :] = zero_v

    def zero_body(k, carry):
        row0 = s * RPT + k * ZB
        pltpu.sync_copy(zbuf, acc.at[pl.ds(row0, ZB)])
        return carry

    lax.fori_loop(0, NZ, zero_body, 0)
    plsc.subcore_barrier()

    def body(g, carry):
        base = pl.multiple_of(s * EPT + g * B, 8)
        pltpu.sync_copy(ii.at[pl.ds(base, B)], ibuf)
        pltpu.sync_copy(jj.at[pl.ds(base, B)], jbuf)
        cp1 = pltpu.async_copy(xh.at[c].at[ibuf], bi, sem1)
        cp2 = pltpu.async_copy(xh.at[c].at[jbuf], bj, sem2)
        cp1.wait()
        cp2.wait()
        for e in range(B):
            vi = bi[e, :]
            vj = bj[e, :]
            bd[e, :] = vi - vj
            bnd[e, :] = vj - vi
        pltpu.sync_copy(bd, acc.at[ibuf], add=True)
        pltpu.sync_copy(bnd, acc.at[jbuf], add=True)
        return carry

    lax.fori_loop(0, NCH, body, 0)
    plsc.subcore_barrier()

    row0 = s * RPT
    pltpu.sync_copy(acc.at[pl.ds(row0, RPT)], out.at[c, pl.ds(row0, RPT)])


def kernel(x, iInd, jInd):
    nb, f1, f2, n = x.shape
    feats = f1 * f2
    half = feats // 2
    xT = x.reshape(feats, n).T
    xa = jnp.pad(xT[:, :half], ((0, NP - n), (0, DH - half)))
    xb = jnp.pad(xT[:, half:], ((0, NP - n), (0, DH - half)))
    xh = jnp.stack([xa, xb])
    out2 = _vector_graph_sc(xh, iInd.astype(jnp.int32), jInd.astype(jnp.int32))
    o = jnp.concatenate([out2[0, :n, :half], out2[1, :n, :half]], axis=1)
    return o.T.reshape(nb, f1, f2, n)


# speedup vs baseline: 387.7242x; 2.9471x over previous
import functools

import jax
import jax.numpy as jnp
from jax import lax
from jax.experimental import pallas as pl
from jax.experimental.pallas import tpu as pltpu
from jax.experimental.pallas import tpu_sc as plsc

N = 100000
NP = 100096
E = 1600000
DH = 16
NC = 2
NS = 16
B = 80
G = 10
GB = G * B
EPT = E // NS
NCH = EPT // B
NG = NCH // G
RPT = NP // NS
ZB = 16
NZ = RPT // ZB

_mesh = plsc.VectorSubcoreMesh(
    core_axis_name="c", subcore_axis_name="s", num_cores=NC, num_subcores=NS
)


@functools.partial(
    pl.kernel,
    out_type=jax.ShapeDtypeStruct((NC, NP, DH), jnp.float32),
    mesh=_mesh,
    scratch_types=[
        pltpu.VMEM((G, B), jnp.int32),
        pltpu.VMEM((G, B), jnp.int32),
        pltpu.VMEM((GB, DH), jnp.float32),
        pltpu.VMEM((GB, DH), jnp.float32),
        pltpu.VMEM((ZB, DH), jnp.float32),
        pltpu.VMEM_SHARED((NP, DH), jnp.float32),
        pltpu.SemaphoreType.DMA,
        pltpu.SemaphoreType.DMA,
        pltpu.SemaphoreType.DMA,
    ],
    compiler_params=pltpu.CompilerParams(use_tc_tiling_on_sc=False),
)
def _vector_graph_sc(xh, ii, jj, out, ibig, jbig, bi, bj, zbuf,
                     acc, semi, semg, sems):
    c = lax.axis_index("c")
    s = lax.axis_index("s")

    zero_v = jnp.zeros((DH,), jnp.float32)
    for r in range(ZB):
        zbuf[r, :] = zero_v

    def zero_body(k, carry):
        row0 = s * RPT + k * ZB
        pltpu.sync_copy(zbuf, acc.at[pl.ds(row0, ZB)])
        return carry

    lax.fori_loop(0, NZ, zero_body, 0)
    plsc.subcore_barrier()

    def body(m, carry):
        row0 = s * NCH + m * G
        cpi = pltpu.async_copy(ii.at[pl.ds(row0, G)], ibig, semi)
        cpj = pltpu.async_copy(jj.at[pl.ds(row0, G)], jbig, semi)
        cpi.wait()
        cpj.wait()
        gds = []
        for t in range(G):
            gds.append(pltpu.async_copy(
                xh.at[c].at[ibig.at[t]], bi.at[pl.ds(t * B, B)], semg))
            gds.append(pltpu.async_copy(
                xh.at[c].at[jbig.at[t]], bj.at[pl.ds(t * B, B)], semg))
        for d in gds:
            d.wait()

        @functools.partial(plsc.parallel_loop, 0, GB, unroll=8)
        def comp(e):
            vi = bi[e, :]
            vj = bj[e, :]
            bi[e, :] = vi - vj
            bj[e, :] = vj - vi

        sds = []
        for t in range(G):
            sds.append(pltpu.async_copy(
                bi.at[pl.ds(t * B, B)], acc.at[ibig.at[t]], sems, add=True))
            sds.append(pltpu.async_copy(
                bj.at[pl.ds(t * B, B)], acc.at[jbig.at[t]], sems, add=True))
        for d in sds:
            d.wait()
        return carry

    lax.fori_loop(0, NG, body, 0)
    plsc.subcore_barrier()

    row0 = s * RPT
    pltpu.sync_copy(acc.at[pl.ds(row0, RPT)], out.at[c, pl.ds(row0, RPT)])


def kernel(x, iInd, jInd):
    nb, f1, f2, n = x.shape
    feats = f1 * f2
    half = feats // 2
    xT = x.reshape(feats, n).T
    xa = jnp.pad(xT[:, :half], ((0, NP - n), (0, DH - half)))
    xb = jnp.pad(xT[:, half:], ((0, NP - n), (0, DH - half)))
    xh = jnp.stack([xa, xb])
    ii2 = iInd.astype(jnp.int32).reshape(E // B, B)
    jj2 = jInd.astype(jnp.int32).reshape(E // B, B)
    out2 = _vector_graph_sc(xh, ii2, jj2)
    o = jnp.concatenate([out2[0, :n, :half], out2[1, :n, :half]], axis=1)
    return o.T.reshape(nb, f1, f2, n)
